# Initial kernel scaffold; baseline (speedup 1.0000x reference)
import jax
import jax.numpy as jnp
from jax.experimental import pallas as pl


def kernel(x, table):
    raise NotImplementedError("write your pallas kernel here")


# speedup vs baseline: 1.5697x; 1.5697x over previous
import functools

import jax
import jax.numpy as jnp
from jax import lax
from jax.experimental import pallas as pl
from jax.experimental.pallas import tpu as pltpu
from jax.experimental.pallas import tpu_sc as plsc

D = 32
CHUNK = 128
NBUF = 4


@functools.lru_cache(maxsize=None)
def _build(n_total: int, nw: int):
    per_w = n_total // nw
    n_chunks = per_w // CHUNK
    ngroups = n_chunks // NBUF
    mesh = plsc.VectorSubcoreMesh(core_axis_name="c", subcore_axis_name="s")

    @functools.partial(
        pl.kernel,
        mesh=mesh,
        out_type=jax.ShapeDtypeStruct((n_total, D), jnp.float32),
        scratch_types=[
            pltpu.VMEM((n_chunks, CHUNK), jnp.int32),
            *[pltpu.VMEM((CHUNK, D), jnp.float32) for _ in range(NBUF)],
            *[pltpu.SemaphoreType.DMA for _ in range(NBUF)],
        ],
        compiler_params=pltpu.CompilerParams(use_tc_tiling_on_sc=False),
    )
    def embed_kernel(idx_hbm, table_hbm, out_hbm, idx_v, *bufs_and_sems):
        rows = bufs_and_sems[:NBUF]
        sems = bufs_and_sems[NBUF : 2 * NBUF]
        wid = lax.axis_index("s") * 2 + lax.axis_index("c")
        base = wid * per_w

        pltpu.sync_copy(idx_hbm.at[wid], idx_v)

        def start(j, b):
            pltpu.async_copy(table_hbm.at[idx_v.at[j]], rows[b], sems[b])

        def finish(j, b):
            pltpu.make_async_copy(table_hbm.at[idx_v.at[j]], rows[b], sems[b]).wait()
            pltpu.sync_copy(rows[b], out_hbm.at[pl.ds(base + j * CHUNK, CHUNK)])

        for b in range(NBUF):
            start(b, b)

        def group(g, carry):
            for b in range(NBUF):
                j = g * NBUF + b
                finish(j, b)
                start(j + NBUF, b)
            return carry

        lax.fori_loop(0, ngroups - 1, group, 0)

        for b in range(NBUF):
            finish((ngroups - 1) * NBUF + b, b)

    return embed_kernel


def kernel(x, table):
    batch, n_fields = x.shape
    n_total = batch * n_fields
    info = plsc.get_sparse_core_info()
    nw = info.num_cores * info.num_subcores
    per_w = n_total // nw
    idx = x.reshape(nw, per_w // CHUNK, CHUNK).astype(jnp.int32)
    out = _build(n_total, nw)(idx, table)
    return out.reshape(batch, n_fields, table.shape[1])


# speedup vs baseline: 1.5741x; 1.0029x over previous
import functools

import jax
import jax.numpy as jnp
from jax import lax
from jax.experimental import pallas as pl
from jax.experimental.pallas import tpu as pltpu
from jax.experimental.pallas import tpu_sc as plsc

D = 32
CHUNK = 128
LOOKAHEAD = 4
NBUF = 8


@functools.lru_cache(maxsize=None)
def _build(n_total: int, nw: int):
    per_w = n_total // nw
    n_chunks = per_w // CHUNK
    assert n_chunks % NBUF == 0 and n_chunks >= 2 * LOOKAHEAD + NBUF
    mesh = plsc.VectorSubcoreMesh(core_axis_name="c", subcore_axis_name="s")

    @functools.partial(
        pl.kernel,
        mesh=mesh,
        out_type=jax.ShapeDtypeStruct((n_total, D), jnp.float32),
        scratch_types=[
            pltpu.VMEM((n_chunks, CHUNK), jnp.int32),
            *[pltpu.VMEM((CHUNK, D), jnp.float32) for _ in range(NBUF)],
            *[pltpu.SemaphoreType.DMA for _ in range(2 * NBUF)],
        ],
        compiler_params=pltpu.CompilerParams(use_tc_tiling_on_sc=False),
    )
    def embed_kernel(idx_hbm, table_hbm, out_hbm, idx_v, *rest):
        rows = rest[:NBUF]
        g_sems = rest[NBUF : 2 * NBUF]
        w_sems = rest[2 * NBUF : 3 * NBUF]
        wid = lax.axis_index("s") * 2 + lax.axis_index("c")
        base = wid * per_w

        pltpu.sync_copy(idx_hbm.at[wid], idx_v)

        def gstart(j, b):
            pltpu.async_copy(table_hbm.at[idx_v.at[j]], rows[b], g_sems[b])

        def gwait(j, b):
            pltpu.make_async_copy(
                table_hbm.at[idx_v.at[j]], rows[b], g_sems[b]
            ).wait()

        def out_at(j):
            return out_hbm.at[pl.ds(base + j * CHUNK, CHUNK)]

        def wstart(j, b):
            pltpu.async_copy(rows[b], out_at(j), w_sems[b])

        def wwait(j, b):
            pltpu.make_async_copy(rows[b], out_at(j), w_sems[b]).wait()

        for j in range(LOOKAHEAD):
            gstart(j, j % NBUF)
        for j in range(LOOKAHEAD):
            gwait(j, j % NBUF)
            wstart(j, j % NBUF)
            gstart(j + LOOKAHEAD, (j + LOOKAHEAD) % NBUF)

        def body(g, carry):
            for u in range(NBUF):
                j = g * NBUF + LOOKAHEAD + u
                b = (LOOKAHEAD + u) % NBUF
                gwait(j, b)
                wstart(j, b)
                lb = u % NBUF
                wwait(j - LOOKAHEAD, lb)
                gstart(j + LOOKAHEAD, lb)
            return carry

        lax.fori_loop(0, (n_chunks - 2 * LOOKAHEAD) // NBUF, body, 0)

        for j in range(n_chunks - LOOKAHEAD, n_chunks):
            gwait(j, j % NBUF)
            wstart(j, j % NBUF)
        for j in range(n_chunks - 2 * LOOKAHEAD, n_chunks):
            wwait(j, j % NBUF)

    return embed_kernel


def kernel(x, table):
    batch, n_fields = x.shape
    n_total = batch * n_fields
    info = plsc.get_sparse_core_info()
    nw = info.num_cores * info.num_subcores
    per_w = n_total // nw
    idx = x.reshape(nw, per_w // CHUNK, CHUNK).astype(jnp.int32)
    out = _build(n_total, nw)(idx, table)
    return out.reshape(batch, n_fields, table.shape[1])
